# Initial kernel scaffold; baseline (speedup 1.0000x reference)
import jax
import jax.numpy as jnp
from jax.experimental import pallas as pl


def kernel(t_query, t0, tau_max, H):
    raise NotImplementedError("write your pallas kernel here")


# speedup vs baseline: 162.3059x; 162.3059x over previous
import functools

import jax
import jax.numpy as jnp
from jax import lax
from jax.experimental import pallas as pl
from jax.experimental.pallas import tpu as pltpu
from jax.experimental.pallas import tpu_sc as plsc

M = 100000
G = 64
Q = 819200

NC = 2
NS = 16
NW = NC * NS
QPW = Q // NW
C = 512
NITER = QPW // C
NGRP = C // 16
NSEG = C // 128

_GATHER_DNUMS = lax.GatherDimensionNumbers(
    offset_dims=(), collapsed_slice_dims=(0,), start_index_map=(0,))


def _lane_bcast(v, lane):
    idx = jnp.full((16, 1), lane, jnp.int32)
    return lax.gather(v, idx, _GATHER_DNUMS, (1,),
                      mode=lax.GatherScatterMode.PROMISE_IN_BOUNDS)


def _body(tq_hbm, params_hbm, h_hbm, out_hbm,
          tq_v, w_v, idxlo_v, idxhi_v, lo_v, hi_v, par_v, sem):
    wid = lax.axis_index("s") * NC + lax.axis_index("c")
    base0 = wid * QPW
    pltpu.sync_copy(params_hbm, par_v)
    T0v = par_v[pl.ds(0, 16)]
    Dv = par_v[pl.ds(16, 16)]
    T1v = par_v[pl.ds(32, 16)]

    def iter_body(k, _):
        base = base0 + k * C
        pltpu.sync_copy(tq_hbm.at[pl.ds(base, C)], tq_v)

        def grp(j, _):
            tq = tq_v[pl.ds(j * 16, 16)]
            tqc = jnp.minimum(jnp.maximum(tq, T0v), T1v)
            pos = (tqc - T0v) / Dv
            idx = pos.astype(jnp.int32)
            t_lo = T0v + idx.astype(jnp.float32) * Dv
            idx = jnp.where(tqc < t_lo, idx - 1, idx)
            t_hi = T0v + (idx + 1).astype(jnp.float32) * Dv
            idx = jnp.where(tqc >= t_hi, idx + 1, idx)
            idx = jnp.clip(idx, 0, M - 2)
            t_lo = T0v + idx.astype(jnp.float32) * Dv
            t_hi = T0v + (idx + 1).astype(jnp.float32) * Dv
            w = (tqc - t_lo) / (t_hi - t_lo)
            w_v[pl.ds(j * 16, 16)] = w
            idxlo_v[pl.ds(j * 16, 16)] = idx
            idxhi_v[pl.ds(j * 16, 16)] = idx + 1
            return 0

        lax.fori_loop(0, NGRP, grp, 0)

        cps = []
        for g in range(NSEG):
            s = pl.ds(g * 128, 128)
            cps.append(pltpu.async_copy(
                h_hbm.at[idxlo_v.at[s]], lo_v.at[s, :], sem))
            cps.append(pltpu.async_copy(
                h_hbm.at[idxhi_v.at[s]], hi_v.at[s, :], sem))
        for cp in cps:
            cp.wait()

        def blend(jq, _):
            wv = w_v[pl.ds(jq * 16, 16)]
            ov = 1.0 - wv
            for c16 in range(16):
                wb = _lane_bcast(wv, c16)
                ob = _lane_bcast(ov, c16)
                c = jq * 16 + c16
                for g in range(G // 16):
                    sl = pl.ds(g * 16, 16)
                    lo_v[c, sl] = ob * lo_v[c, sl] + wb * hi_v[c, sl]
            return 0

        lax.fori_loop(0, NGRP, blend, 0)
        pltpu.sync_copy(lo_v, out_hbm.at[pl.ds(base, C)])
        return 0

    lax.fori_loop(0, NITER, iter_body, 0)


@functools.partial(jax.jit, static_argnames=())
def _interp(t_query, params, H):
    mesh = plsc.VectorSubcoreMesh(core_axis_name="c", subcore_axis_name="s")
    f = pl.kernel(
        _body,
        mesh=mesh,
        compiler_params=pltpu.CompilerParams(use_tc_tiling_on_sc=False),
        out_type=jax.ShapeDtypeStruct((Q, G), jnp.float32),
        scratch_types=[
            pltpu.VMEM((C,), jnp.float32),
            pltpu.VMEM((C,), jnp.float32),
            pltpu.VMEM((C,), jnp.int32),
            pltpu.VMEM((C,), jnp.int32),
            pltpu.VMEM((C, G), jnp.float32),
            pltpu.VMEM((C, G), jnp.float32),
            pltpu.VMEM((48,), jnp.float32),
            pltpu.SemaphoreType.DMA,
        ],
    )
    return f(t_query, params, H)


def kernel(t_query, t0, tau_max, H):
    t0 = jnp.asarray(t0, jnp.float32)
    T0 = (t0 - tau_max).astype(jnp.float32)
    delta = ((t0 - T0) / jnp.float32(M - 1)).astype(jnp.float32)
    T1 = T0 + jnp.float32(M - 1) * delta
    params = jnp.concatenate([
        jnp.broadcast_to(T0, (16,)),
        jnp.broadcast_to(delta, (16,)),
        jnp.broadcast_to(T1, (16,)),
    ]).astype(jnp.float32)
    return _interp(t_query, params, H)


# speedup vs baseline: 208.7011x; 1.2859x over previous
import functools

import jax
import jax.numpy as jnp
from jax import lax
from jax.experimental import pallas as pl
from jax.experimental.pallas import tpu as pltpu
from jax.experimental.pallas import tpu_sc as plsc

M = 100000
G = 64
Q = 819200

NC = 2
NS = 16
NW = NC * NS
QPW = Q // NW
C = 128
NITER = QPW // C
NGRP = C // 16
NSEG = C // 128

_GATHER_DNUMS = lax.GatherDimensionNumbers(
    offset_dims=(), collapsed_slice_dims=(0,), start_index_map=(0,))


def _lane_bcast(v, lane):
    idx = jnp.full((16, 1), lane, jnp.int32)
    return lax.gather(v, idx, _GATHER_DNUMS, (1,),
                      mode=lax.GatherScatterMode.PROMISE_IN_BOUNDS)


def _body(tq_hbm, params_hbm, d_hbm, out_hbm,
          tq_all, par_v,
          w0, w1, i0, i1, d0, d1, o0, o1,
          g0, g1, s0, s1):
    ws = [w0, w1]
    idxs = [i0, i1]
    rows = [d0, d1]
    outs = [o0, o1]
    gsems = [g0, g1]
    osems = [s0, s1]

    wid = lax.axis_index("s") * NC + lax.axis_index("c")
    base0 = wid * QPW
    pltpu.sync_copy(params_hbm, par_v)
    pltpu.sync_copy(tq_hbm.at[pl.ds(base0, QPW)], tq_all)
    T0v = par_v[pl.ds(0, 16)]
    Dv = par_v[pl.ds(16, 16)]
    T1v = par_v[pl.ds(32, 16)]

    def start(b, k):
        def grp(j, _):
            tq = tq_all[pl.ds(k * C + j * 16, 16)]
            tqc = jnp.minimum(jnp.maximum(tq, T0v), T1v)
            pos = (tqc - T0v) / Dv
            idx = pos.astype(jnp.int32)
            t_lo = T0v + idx.astype(jnp.float32) * Dv
            idx = jnp.where(tqc < t_lo, idx - 1, idx)
            t_hi = T0v + (idx + 1).astype(jnp.float32) * Dv
            idx = jnp.where(tqc >= t_hi, idx + 1, idx)
            idx = jnp.clip(idx, 0, M - 2)
            t_lo = T0v + idx.astype(jnp.float32) * Dv
            t_hi = T0v + (idx + 1).astype(jnp.float32) * Dv
            w = (tqc - t_lo) / (t_hi - t_lo)
            ws[b][pl.ds(j * 16, 16)] = w
            idxs[b][pl.ds(j * 16, 16)] = idx
            return 0

        lax.fori_loop(0, NGRP, grp, 0)
        for seg in range(NSEG):
            s = pl.ds(seg * 128, 128)
            pltpu.async_copy(d_hbm.at[idxs[b].at[s]], rows[b].at[s, :],
                             gsems[b])

    def drain_gather(b):
        for seg in range(NSEG):
            s = pl.ds(seg * 128, 128)
            pltpu.make_async_copy(d_hbm.at[pl.ds(0, 128), :],
                                  rows[b].at[s, :], gsems[b]).wait()

    def drain_out(b):
        pltpu.make_async_copy(outs[b], out_hbm.at[pl.ds(0, C)],
                              osems[b]).wait()

    def finish(b, k):
        drain_gather(b)

        def blend(jq, _):
            wv = ws[b][pl.ds(jq * 16, 16)]
            ov = 1.0 - wv
            for c16 in range(16):
                wb = _lane_bcast(wv, c16)
                ob = _lane_bcast(ov, c16)
                c = jq * 16 + c16
                for g in range(G // 16):
                    sl = pl.ds(g * 16, 16)
                    sh = pl.ds(G + g * 16, 16)
                    outs[b][c, sl] = (ob * rows[b][c, sl]
                                      + wb * rows[b][c, sh])
            return 0

        lax.fori_loop(0, NGRP, blend, 0)
        pltpu.async_copy(outs[b], out_hbm.at[pl.ds(base0 + k * C, C)],
                         osems[b])

    start(0, 0)

    def pair(k2, _):
        k = k2 * 2
        start(1, k + 1)

        @pl.when(k2 > 0)
        def _():
            drain_out(0)

        finish(0, k)

        @pl.when(k + 2 < NITER)
        def _():
            start(0, k + 2)

        @pl.when(k2 > 0)
        def _():
            drain_out(1)

        finish(1, k + 1)
        return 0

    lax.fori_loop(0, NITER // 2, pair, 0)
    drain_out(0)
    drain_out(1)


@jax.jit
def _interp(t_query, params, D):
    mesh = plsc.VectorSubcoreMesh(core_axis_name="c", subcore_axis_name="s")
    f = pl.kernel(
        _body,
        mesh=mesh,
        out_type=jax.ShapeDtypeStruct((Q, G), jnp.float32),
        scratch_types=[
            pltpu.VMEM((QPW,), jnp.float32),
            pltpu.VMEM((128,), jnp.float32),
            pltpu.VMEM((C,), jnp.float32),
            pltpu.VMEM((C,), jnp.float32),
            pltpu.VMEM((C,), jnp.int32),
            pltpu.VMEM((C,), jnp.int32),
            pltpu.VMEM((C, 2 * G), jnp.float32),
            pltpu.VMEM((C, 2 * G), jnp.float32),
            pltpu.VMEM((C, G), jnp.float32),
            pltpu.VMEM((C, G), jnp.float32),
            pltpu.SemaphoreType.DMA,
            pltpu.SemaphoreType.DMA,
            pltpu.SemaphoreType.DMA,
            pltpu.SemaphoreType.DMA,
        ],
    )
    return f(t_query, params, D)


def kernel(t_query, t0, tau_max, H):
    t0 = jnp.asarray(t0, jnp.float32)
    T0 = (t0 - tau_max).astype(jnp.float32)
    delta = ((t0 - T0) / jnp.float32(M - 1)).astype(jnp.float32)
    T1 = T0 + jnp.float32(M - 1) * delta
    params = jnp.concatenate([
        jnp.broadcast_to(T0, (16,)),
        jnp.broadcast_to(delta, (16,)),
        jnp.broadcast_to(T1, (16,)),
        jnp.zeros((80,), jnp.float32),
    ]).astype(jnp.float32)
    D = jnp.concatenate([H[:-1], H[1:]], axis=1)
    return _interp(t_query, params, D)


# speedup vs baseline: 210.9313x; 1.0107x over previous
import functools

import jax
import jax.numpy as jnp
from jax import lax
from jax.experimental import pallas as pl
from jax.experimental.pallas import tpu as pltpu
from jax.experimental.pallas import tpu_sc as plsc

M = 100000
G = 64
Q = 819200

NC = 2
NS = 16
NW = NC * NS
QPW = Q // NW
C = 160
NITER = QPW // C
NGRP = C // 16
SEGS = [(0, 128), (128, 32)]

RB = 2000

_GATHER_DNUMS = lax.GatherDimensionNumbers(
    offset_dims=(), collapsed_slice_dims=(0,), start_index_map=(0,))


def _lane_bcast(v, lane):
    idx = jnp.full((16, 1), lane, jnp.int32)
    return lax.gather(v, idx, _GATHER_DNUMS, (1,),
                      mode=lax.GatherScatterMode.PROMISE_IN_BOUNDS)


def _dbuild_body(hp_hbm, d_ref, buf, sem):
    i = pl.program_id(0)
    cp = pltpu.make_async_copy(hp_hbm.at[pl.ds(i * RB, RB + 8)], buf, sem)
    cp.start()
    cp.wait()
    b = buf[...]
    d_ref[:, 0:G] = b[0:RB, :]
    d_ref[:, G:2 * G] = b[1:RB + 1, :]


def _sc_body(tq_hbm, params_hbm, d_hbm, out_hbm,
             tq_all, par_v,
             w0, w1, i0, i1, d0, d1, o0, o1,
             g0, g1, s0, s1):
    ws = [w0, w1]
    idxs = [i0, i1]
    rows = [d0, d1]
    outs = [o0, o1]
    gsems = [g0, g1]
    osems = [s0, s1]

    wid = lax.axis_index("s") * NC + lax.axis_index("c")
    base0 = wid * QPW
    pltpu.sync_copy(params_hbm, par_v)
    pltpu.sync_copy(tq_hbm.at[pl.ds(base0, QPW)], tq_all)
    T0v = par_v[pl.ds(0, 16)]
    Dv = par_v[pl.ds(16, 16)]
    T1v = par_v[pl.ds(32, 16)]

    def start(b, k):
        def grp(j, _):
            tq = tq_all[pl.ds(k * C + j * 16, 16)]
            tqc = jnp.minimum(jnp.maximum(tq, T0v), T1v)
            pos = (tqc - T0v) / Dv
            idx = pos.astype(jnp.int32)
            t_lo = T0v + idx.astype(jnp.float32) * Dv
            idx = jnp.where(tqc < t_lo, idx - 1, idx)
            t_hi = T0v + (idx + 1).astype(jnp.float32) * Dv
            idx = jnp.where(tqc >= t_hi, idx + 1, idx)
            idx = jnp.clip(idx, 0, M - 2)
            t_lo = T0v + idx.astype(jnp.float32) * Dv
            t_hi = T0v + (idx + 1).astype(jnp.float32) * Dv
            w = (tqc - t_lo) / (t_hi - t_lo)
            ws[b][pl.ds(j * 16, 16)] = w
            idxs[b][pl.ds(j * 16, 16)] = idx
            return 0

        lax.fori_loop(0, NGRP, grp, 0)
        for off, n in SEGS:
            pltpu.async_copy(d_hbm.at[idxs[b].at[pl.ds(off, n)]],
                             rows[b].at[pl.ds(off, n), :], gsems[b])

    def drain_gather(b):
        for off, n in SEGS:
            pltpu.make_async_copy(d_hbm.at[pl.ds(0, n), :],
                                  rows[b].at[pl.ds(off, n), :],
                                  gsems[b]).wait()

    def drain_out(b):
        pltpu.make_async_copy(outs[b], out_hbm.at[pl.ds(0, C)],
                              osems[b]).wait()

    def finish(b, k):
        drain_gather(b)

        def blend(jq, _):
            wv = ws[b][pl.ds(jq * 16, 16)]
            ov = 1.0 - wv
            for c16 in range(16):
                wb = _lane_bcast(wv, c16)
                ob = _lane_bcast(ov, c16)
                c = jq * 16 + c16
                for g in range(G // 16):
                    sl = pl.ds(g * 16, 16)
                    sh = pl.ds(G + g * 16, 16)
                    outs[b][c, sl] = (ob * rows[b][c, sl]
                                      + wb * rows[b][c, sh])
            return 0

        lax.fori_loop(0, NGRP, blend, 0)
        pltpu.async_copy(outs[b], out_hbm.at[pl.ds(base0 + k * C, C)],
                         osems[b])

    start(0, 0)

    def pair(k2, _):
        k = k2 * 2
        start(1, k + 1)

        @pl.when(k2 > 0)
        def _():
            drain_out(0)

        finish(0, k)

        @pl.when(k + 2 < NITER)
        def _():
            start(0, k + 2)

        @pl.when(k2 > 0)
        def _():
            drain_out(1)

        finish(1, k + 1)
        return 0

    lax.fori_loop(0, NITER // 2, pair, 0)
    drain_out(0)
    drain_out(1)


@jax.jit
def _interp(t_query, params, H):
    hpad = jnp.pad(H, ((0, 8), (0, 0)))
    dtab = pl.pallas_call(
        _dbuild_body,
        grid=(M // RB,),
        in_specs=[pl.BlockSpec(memory_space=pl.ANY)],
        out_specs=pl.BlockSpec((RB, 2 * G), lambda i: (i, 0)),
        out_shape=jax.ShapeDtypeStruct((M, 2 * G), jnp.float32),
        scratch_shapes=[pltpu.VMEM((RB + 8, G), jnp.float32),
                        pltpu.SemaphoreType.DMA],
    )(hpad)

    mesh = plsc.VectorSubcoreMesh(core_axis_name="c", subcore_axis_name="s")
    f = pl.kernel(
        _sc_body,
        mesh=mesh,
        out_type=jax.ShapeDtypeStruct((Q, G), jnp.float32),
        scratch_types=[
            pltpu.VMEM((QPW,), jnp.float32),
            pltpu.VMEM((48,), jnp.float32),
            pltpu.VMEM((C,), jnp.float32),
            pltpu.VMEM((C,), jnp.float32),
            pltpu.VMEM((C,), jnp.int32),
            pltpu.VMEM((C,), jnp.int32),
            pltpu.VMEM((C, 2 * G), jnp.float32),
            pltpu.VMEM((C, 2 * G), jnp.float32),
            pltpu.VMEM((C, G), jnp.float32),
            pltpu.VMEM((C, G), jnp.float32),
            pltpu.SemaphoreType.DMA,
            pltpu.SemaphoreType.DMA,
            pltpu.SemaphoreType.DMA,
            pltpu.SemaphoreType.DMA,
        ],
    )
    return f(t_query, params, dtab)


def kernel(t_query, t0, tau_max, H):
    t0 = jnp.asarray(t0, jnp.float32)
    T0 = (t0 - tau_max).astype(jnp.float32)
    delta = ((t0 - T0) / jnp.float32(M - 1)).astype(jnp.float32)
    T1 = T0 + jnp.float32(M - 1) * delta
    params = jnp.concatenate([
        jnp.broadcast_to(T0, (16,)),
        jnp.broadcast_to(delta, (16,)),
        jnp.broadcast_to(T1, (16,)),
    ]).astype(jnp.float32)
    return _interp(t_query, params, H)
